# baseline (speedup 1.0000x reference)
import functools

import numpy as np

import jax
import jax.numpy as jnp
from jax.experimental import pallas as pl
from jax.experimental.pallas import tpu as pltpu

_BN_S = float(1.0 / np.sqrt(1.0 + 1e-5))
_K = 5
_N = 200
_G = 4


def _leaky(x, s):
    return jnp.where(x >= 0, x, s * x)


def _layer_norm(x, eps=1e-5):
    mu = jnp.mean(x, axis=-1, keepdims=True)
    var = jnp.mean((x - mu) ** 2, axis=-1, keepdims=True)
    return (x - mu) / jnp.sqrt(var + eps)


def _mm_nt(a, b):
    return jax.lax.dot_general(a, b, (((1,), (1,)), ((), ())),
                               preferred_element_type=jnp.float32)


def _stack_kernel(x_ref, dw_ref, gw_ref, up0, up1, up2, up3, up4, enc_ref,
                  out_ref):
    ups = (up0, up1, up2, up3, up4)
    n = _N
    row = jax.lax.broadcasted_iota(jnp.int32, (n, n), 0)
    rowk = jax.lax.broadcasted_iota(jnp.int32, (n, _K * n), 0)
    hs = [x_ref[m] for m in range(_G)]
    hcats = [[] for _ in range(_G)]
    for layer in range(5):
        up = ups[layer][...]
        feats = []
        for m in range(_G):
            ht = hs[m]
            g = jax.lax.dot_general(ht, ht, (((0,), (0,)), ((), ())),
                                    preferred_element_type=jnp.float32)
            xxr = jnp.sum(ht * ht, axis=0, keepdims=True)
            pd = 2.0 * g - (xxr + xxr.T)
            vals = pd
            iks = []
            mxs = []
            for _ in range(_K):
                mx = jnp.max(vals, axis=0, keepdims=True)
                cand = jnp.where(vals == mx, row, n)
                ik = jnp.min(cand, axis=0, keepdims=True)
                iks.append(ik)
                mxs.append(mx)
                vals = jnp.where(row == ik, -jnp.inf, vals)
            dist_t = -jnp.concatenate(mxs, axis=0)
            ikcat = jnp.concatenate(iks, axis=1)
            ohcat = (rowk == ikcat).astype(jnp.float32)
            gfa = jax.lax.dot_general(ht, ohcat, (((1,), (0,)), ((), ())),
                                      preferred_element_type=jnp.float32)
            gf0 = gfa[:, :n]
            sub_t = jnp.concatenate(
                [jnp.sum(gfa[:, k * n:(k + 1) * n] * gf0, axis=0,
                         keepdims=True) for k in range(_K)],
                axis=0)
            s2 = sub_t * sub_t
            nrm2 = jax.lax.dot_general(s2, s2, (((1,), (1,)), ((), ())),
                                       preferred_element_type=jnp.float32)
            nrm = jnp.maximum(jnp.sqrt(nrm2), 1e-12)
            gw = gw_ref[layer:layer + 1, :]
            c = gw / nrm
            t = jnp.dot(c, sub_t, preferred_element_type=jnp.float32)
            w2 = jax.nn.sigmoid(_BN_S * sub_t * t)
            dw = dw_ref[0:1, layer:layer + 1]
            w1 = jax.nn.sigmoid(_BN_S * dist_t * dw)
            w = w1 * w2
            wcat = jnp.concatenate(
                [w[k:k + 1, :] for k in range(_K)], axis=1)
            hc = ht[3:] if layer == 0 else ht
            gfa_c = gfa[3:] if layer == 0 else gfa
            hc_t = jnp.concatenate([hc] * _K, axis=1)
            feats.append(wcat * gfa_c + (1.0 - wcat) * hc_t)
        fcat = jnp.concatenate(feats, axis=1)
        y = _leaky(_BN_S * jnp.dot(up, fcat,
                                   preferred_element_type=jnp.float32),
                   0.02)
        for m in range(_G):
            base = m * _K * n
            acc = y[:, base:base + n]
            for k in range(1, _K):
                acc = acc + y[:, base + k * n:base + (k + 1) * n]
            hs[m] = acc * (1.0 / _K)
            hcats[m].append(hs[m])
    ecat = jnp.concatenate(
        [jnp.concatenate(hcats[m], axis=0) for m in range(_G)],
        axis=1)
    e = _leaky(_BN_S * jax.lax.dot_general(
        ecat, enc_ref[...], (((0,), (1,)), ((), ())),
        preferred_element_type=jnp.float32), 0.2)
    emb = e.shape[1]
    for m in range(_G):
        em = e[m * n:(m + 1) * n]
        out_ref[m, :, :emb] = jnp.max(em, axis=0, keepdims=True)
        out_ref[m, :, emb:] = jnp.mean(em, axis=0, keepdims=True)


def _stack_call(x, dist_w, gm_w, up_ws, enc_w):
    b = x.shape[0]
    emb = enc_w.shape[0]
    full = lambda a: pl.BlockSpec(a.shape, lambda i: (0,) * a.ndim)
    return pl.pallas_call(
        _stack_kernel,
        grid=(b // _G,),
        in_specs=[
            pl.BlockSpec((_G, x.shape[1], x.shape[2]),
                         lambda i: (i, 0, 0)),
            full(dist_w), full(gm_w),
            *[full(u) for u in up_ws], full(enc_w),
        ],
        out_specs=pl.BlockSpec((_G, 1, 2 * emb), lambda i: (i, 0, 0)),
        out_shape=jax.ShapeDtypeStruct((b, 1, 2 * emb), jnp.float32),
        compiler_params=pltpu.CompilerParams(
            dimension_semantics=("arbitrary",)),
    )(x, dist_w, gm_w, *up_ws, enc_w).reshape(b, 2 * emb)


def _linear_kernel(x_ref, w_ref, *rest, nk, scale, ln, slope, has_res,
                   has_bias):
    out_ref = rest[-1]
    i = 0
    res_ref = bias_ref = None
    if has_res:
        res_ref = rest[i]
        i += 1
    if has_bias:
        bias_ref = rest[i]
    k = pl.program_id(0)

    @pl.when(k == 0)
    def _init():
        out_ref[...] = jnp.zeros_like(out_ref)

    out_ref[...] += _mm_nt(x_ref[...], w_ref[...])

    @pl.when(k == nk - 1)
    def _fin():
        h = out_ref[...]
        if scale != 1.0:
            h = scale * h
        if ln:
            h = _layer_norm(h)
        if has_bias:
            h = h + bias_ref[...]
        if has_res:
            h = h + res_ref[...]
        if slope is not None:
            h = _leaky(h, slope)
        out_ref[...] = h


def _linear_call(x, w, *, scale=1.0, ln=False, slope=None, res=None,
                 bias=None):
    b, d_in = x.shape
    d_out = w.shape[0]
    tile = 1024 if d_in % 1024 == 0 else d_in
    nk = d_in // tile
    operands = [x, w]
    in_specs = [
        pl.BlockSpec((b, tile), lambda k: (0, k)),
        pl.BlockSpec((d_out, tile), lambda k: (0, k)),
    ]
    if res is not None:
        operands.append(res)
        in_specs.append(pl.BlockSpec((b, d_out), lambda k: (0, 0)))
    if bias is not None:
        operands.append(bias.reshape(1, d_out))
        in_specs.append(pl.BlockSpec((1, d_out), lambda k: (0, 0)))
    fn = functools.partial(_linear_kernel, nk=nk, scale=scale, ln=ln,
                           slope=slope, has_res=res is not None,
                           has_bias=bias is not None)
    return pl.pallas_call(
        fn,
        grid=(nk,),
        in_specs=in_specs,
        out_specs=pl.BlockSpec((b, d_out), lambda k: (0, 0)),
        out_shape=jax.ShapeDtypeStruct((b, d_out), jnp.float32),
        compiler_params=pltpu.CompilerParams(
            dimension_semantics=("arbitrary",)),
    )(*operands)


def kernel(x, env, idx_base, params):
    del idx_base
    dist_w = jnp.stack(params['dist_w']).reshape(1, 5)
    gm_w = jnp.stack(params['gm_w'])
    pooled = _stack_call(x, dist_w, gm_w, params['up_w'],
                         params['enc_conv_w'])

    h = _linear_call(pooled, params['merge_w'], scale=_BN_S, slope=0.2)
    h = jnp.concatenate([h, env[:, None]], axis=1)
    for blk in params['blocks']:
        identity = h
        d_in = identity.shape[1]
        h = _linear_call(h, blk['w1'], ln=True, slope=0.2)
        h = _linear_call(h, blk['w2'], ln=True, slope=0.2)
        d_out = blk['w3'].shape[0]
        idx = (np.arange(d_out) * d_in) // d_out
        res = identity[:, idx]
        h = _linear_call(h, blk['w3'], ln=True, res=res, slope=0.2)
    return _linear_call(h, params['fc_w'], bias=params['fc_b'])


# speedup vs baseline: 1.3607x; 1.3607x over previous
import functools

import numpy as np

import jax
import jax.numpy as jnp
from jax.experimental import pallas as pl
from jax.experimental.pallas import tpu as pltpu

_BN_S = float(1.0 / np.sqrt(1.0 + 1e-5))
_K = 5
_N = 200
_G = 4


def _leaky(x, s):
    return jnp.where(x >= 0, x, s * x)


def _layer_norm(x, eps=1e-5):
    mu = jnp.mean(x, axis=-1, keepdims=True)
    var = jnp.mean((x - mu) ** 2, axis=-1, keepdims=True)
    return (x - mu) / jnp.sqrt(var + eps)


def _mm_nt(a, b):
    return jax.lax.dot_general(a, b, (((1,), (1,)), ((), ())),
                               preferred_element_type=jnp.float32)


_NS = 256


def _stack_kernel(x_ref, dw_ref, gw_ref, up0, up1, up2, up3, up4, enc_ref,
                  out_ref):
    ups = (up0, up1, up2, up3, up4)
    n, ns = _N, _NS
    row = jax.lax.broadcasted_iota(jnp.int32, (n, n), 0)
    rowk = jax.lax.broadcasted_iota(jnp.int32, (n, _K * ns), 0)
    zpad = jnp.zeros((1, ns - n), jnp.float32)
    ipad = jnp.full((1, ns - n), n, jnp.int32)
    hs = [x_ref[m] for m in range(_G)]
    hcats = [[] for _ in range(_G)]
    for layer in range(5):
        up = ups[layer][...]
        for m in range(_G):
            ht = hs[m]
            htc = ht[:, :n]
            g = jax.lax.dot_general(htc, htc, (((0,), (0,)), ((), ())),
                                    preferred_element_type=jnp.float32)
            xxr = jnp.sum(htc * htc, axis=0, keepdims=True)
            pd = 2.0 * g - (xxr + xxr.T)
            vals = pd
            iks = []
            mxs = []
            for _ in range(_K):
                mx = jnp.max(vals, axis=0, keepdims=True)
                cand = jnp.where(vals == mx, row, n)
                ik = jnp.min(cand, axis=0, keepdims=True)
                iks.append(ik)
                mxs.append(mx)
                vals = jnp.where(row == ik, -jnp.inf, vals)
            ikcat = jnp.concatenate(
                [p for ik in iks for p in (ik, ipad)], axis=1)
            ohcat = (rowk == ikcat).astype(jnp.float32)
            gfa = jax.lax.dot_general(htc, ohcat, (((1,), (0,)), ((), ())),
                                      preferred_element_type=jnp.float32)
            gf0 = gfa[:, :ns]
            sub_t = jnp.concatenate(
                [jnp.sum(gfa[:, k * ns:(k + 1) * ns] * gf0, axis=0,
                         keepdims=True) for k in range(_K)],
                axis=0)
            s2 = sub_t * sub_t
            nrm2 = jax.lax.dot_general(s2, s2, (((1,), (1,)), ((), ())),
                                       preferred_element_type=jnp.float32)
            nrm = jnp.maximum(jnp.sqrt(nrm2), 1e-12)
            gw = gw_ref[layer:layer + 1, :]
            c = gw / nrm
            t = jnp.dot(c, sub_t, preferred_element_type=jnp.float32)
            w2 = jax.nn.sigmoid(_BN_S * sub_t * t)
            dist_t = -jnp.concatenate(
                [p for mx in mxs for p in (mx, zpad)], axis=1)
            dw = dw_ref[0:1, layer:layer + 1]
            w1 = jax.nn.sigmoid(_BN_S * dist_t * dw)
            w2cat = jnp.concatenate(
                [w2[k:k + 1, :] for k in range(_K)], axis=1)
            wcat = w1 * w2cat
            hc = ht[3:] if layer == 0 else ht
            gfa_c = gfa[3:] if layer == 0 else gfa
            hc_t = jnp.concatenate([hc] * _K, axis=1)
            feat = wcat * gfa_c + (1.0 - wcat) * hc_t
            y = _leaky(_BN_S * jnp.dot(up, feat,
                                       preferred_element_type=jnp.float32),
                       0.02)
            acc = y[:, :ns]
            for k in range(1, _K):
                acc = acc + y[:, k * ns:(k + 1) * ns]
            hs[m] = acc * (1.0 / _K)
            hcats[m].append(hs[m])
    enc_b = enc_ref[...].astype(jnp.bfloat16)
    emb = enc_b.shape[0]
    for m in range(_G):
        hcat_m = jnp.concatenate(hcats[m], axis=0)
        e = _leaky(_BN_S * jax.lax.dot_general(
            hcat_m.astype(jnp.bfloat16), enc_b, (((0,), (1,)), ((), ())),
            preferred_element_type=jnp.float32), 0.2)
        em = e[:n]
        out_ref[m, :, :emb] = jnp.max(em, axis=0, keepdims=True)
        out_ref[m, :, emb:] = jnp.mean(em, axis=0, keepdims=True)


def _stack_call(x, dist_w, gm_w, up_ws, enc_w):
    x = jnp.pad(x, ((0, 0), (0, 0), (0, _NS - x.shape[2])))
    b = x.shape[0]
    emb = enc_w.shape[0]
    full = lambda a: pl.BlockSpec(a.shape, lambda i: (0,) * a.ndim)
    return pl.pallas_call(
        _stack_kernel,
        grid=(b // _G,),
        in_specs=[
            pl.BlockSpec((_G, x.shape[1], x.shape[2]),
                         lambda i: (i, 0, 0)),
            full(dist_w), full(gm_w),
            *[full(u) for u in up_ws], full(enc_w),
        ],
        out_specs=pl.BlockSpec((_G, 1, 2 * emb), lambda i: (i, 0, 0)),
        out_shape=jax.ShapeDtypeStruct((b, 1, 2 * emb), jnp.float32),
        compiler_params=pltpu.CompilerParams(
            dimension_semantics=("arbitrary",)),
    )(x, dist_w, gm_w, *up_ws, enc_w).reshape(b, 2 * emb)


def _dec_b0_kernel(x_ref, mw_ref, w1_ref, env_ref, w2_ref, w3_ref, out_ref):
    m = _leaky(_BN_S * _mm_nt(x_ref[...], mw_ref[...]), 0.2)
    d = w1_ref.shape[1] - 1
    raw1 = _mm_nt(m, w1_ref[:, :d]) + jax.lax.dot_general(
        env_ref[...], w1_ref[:, d:], (((1,), (1,)), ((), ())),
        preferred_element_type=jnp.float32)
    x2 = _leaky(_layer_norm(raw1), 0.2)
    x3 = _leaky(_layer_norm(_mm_nt(x2, w2_ref[...])), 0.2)
    out_ref[...] = _leaky(_layer_norm(_mm_nt(x3, w3_ref[...])) + m, 0.2)


def _dec_b1_kernel(x_ref, w4_ref, w5_ref, w6_ref, out_ref):
    h = x_ref[...]
    x5 = _leaky(_layer_norm(_mm_nt(h, w4_ref[...])), 0.2)
    x6 = _leaky(_layer_norm(_mm_nt(x5, w5_ref[...])), 0.2)
    z = _layer_norm(_mm_nt(x6, w6_ref[...]))
    d_in, d_out = h.shape[1], z.shape[1]
    ri = jax.lax.broadcasted_iota(jnp.int32, (d_in, d_out), 0)
    cj = jax.lax.broadcasted_iota(jnp.int32, (d_in, d_out), 1)
    rmat = (ri == cj // 2).astype(jnp.float32)
    res = jax.lax.dot_general(h, rmat, (((1,), (0,)), ((), ())),
                              preferred_element_type=jnp.float32)
    out_ref[...] = _leaky(z + res, 0.2)


def _dec_b2_kernel(x_ref, w7_ref, w8_ref, w9_ref, out_ref):
    h = x_ref[...]
    x8 = _leaky(_layer_norm(_mm_nt(h, w7_ref[...])), 0.2)
    x9 = _leaky(_layer_norm(_mm_nt(x8, w8_ref[...])), 0.2)
    out_ref[...] = _leaky(_layer_norm(_mm_nt(x9, w9_ref[...])) + h, 0.2)


def _dec_call(fn, operands, d_out):
    b = operands[0].shape[0]
    return pl.pallas_call(
        fn,
        out_shape=jax.ShapeDtypeStruct((b, d_out), jnp.float32),
    )(*operands)


def _linear_kernel(x_ref, w_ref, *rest, nk, scale, ln, slope, has_res,
                   has_bias):
    out_ref = rest[-1]
    i = 0
    res_ref = bias_ref = None
    if has_res:
        res_ref = rest[i]
        i += 1
    if has_bias:
        bias_ref = rest[i]
    k = pl.program_id(0)

    @pl.when(k == 0)
    def _init():
        out_ref[...] = jnp.zeros_like(out_ref)

    out_ref[...] += _mm_nt(x_ref[...], w_ref[...])

    @pl.when(k == nk - 1)
    def _fin():
        h = out_ref[...]
        if scale != 1.0:
            h = scale * h
        if ln:
            h = _layer_norm(h)
        if has_bias:
            h = h + bias_ref[...]
        if has_res:
            h = h + res_ref[...]
        if slope is not None:
            h = _leaky(h, slope)
        out_ref[...] = h


def _linear_call(x, w, *, scale=1.0, ln=False, slope=None, res=None,
                 bias=None):
    b, d_in = x.shape
    d_out = w.shape[0]
    tile = 1024 if d_in % 1024 == 0 else d_in
    nk = d_in // tile
    operands = [x, w]
    in_specs = [
        pl.BlockSpec((b, tile), lambda k: (0, k)),
        pl.BlockSpec((d_out, tile), lambda k: (0, k)),
    ]
    if res is not None:
        operands.append(res)
        in_specs.append(pl.BlockSpec((b, d_out), lambda k: (0, 0)))
    if bias is not None:
        operands.append(bias.reshape(1, d_out))
        in_specs.append(pl.BlockSpec((1, d_out), lambda k: (0, 0)))
    fn = functools.partial(_linear_kernel, nk=nk, scale=scale, ln=ln,
                           slope=slope, has_res=res is not None,
                           has_bias=bias is not None)
    return pl.pallas_call(
        fn,
        grid=(nk,),
        in_specs=in_specs,
        out_specs=pl.BlockSpec((b, d_out), lambda k: (0, 0)),
        out_shape=jax.ShapeDtypeStruct((b, d_out), jnp.float32),
        compiler_params=pltpu.CompilerParams(
            dimension_semantics=("arbitrary",)),
    )(*operands)


def kernel(x, env, idx_base, params):
    del idx_base
    dist_w = jnp.stack(params['dist_w']).reshape(1, 5)
    gm_w = jnp.stack(params['gm_w'])
    pooled = _stack_call(x, dist_w, gm_w, params['up_w'],
                         params['enc_conv_w'])

    b0, b1, b2 = params['blocks']
    h = _dec_call(_dec_b0_kernel,
                  [pooled, params['merge_w'], b0['w1'], env[:, None],
                   b0['w2'], b0['w3']], b0['w3'].shape[0])
    h = _dec_call(_dec_b1_kernel, [h, b1['w1'], b1['w2'], b1['w3']],
                  b1['w3'].shape[0])
    h = _dec_call(_dec_b2_kernel, [h, b2['w1'], b2['w2'], b2['w3']],
                  b2['w3'].shape[0])
    return _linear_call(h, params['fc_w'], bias=params['fc_b'])
